# baseline (speedup 1.0000x reference)
import functools

import jax
import jax.numpy as jnp
from jax import lax
from jax.experimental import pallas as pl
from jax.experimental.pallas import tpu as pltpu
from jax.experimental.pallas import tpu_sc as plsc

_EMBED = 128
_LANES = 16
_IPAD = 8

_mesh = plsc.VectorSubcoreMesh(core_axis_name="c", subcore_axis_name="s")


@functools.partial(
    pl.kernel,
    out_type=jax.ShapeDtypeStruct((_LANES,), jnp.float32),
    mesh=_mesh,
    compiler_params=pltpu.CompilerParams(needs_layout_passes=False),
    scratch_types=[
        pltpu.VMEM((_IPAD,), jnp.int32),
        pltpu.VMEM((_IPAD,), jnp.int32),
        pltpu.VMEM((_IPAD, _EMBED), jnp.float32),
        pltpu.VMEM((_IPAD, _EMBED), jnp.float32),
        pltpu.VMEM((_LANES,), jnp.float32),
        pltpu.VMEM((_LANES,), jnp.float32),
        pltpu.SemaphoreType.DMA,
        pltpu.SemaphoreType.DMA,
    ],
)
def _sc_skipgram(ti_hbm, ci_hbm, w_hbm, c_hbm, out_hbm,
                 ti_v, ci_v, wrow_v, crow_v, red_v, out_v, sem_w, sem_c):
    wid = lax.axis_index("s") * 2 + lax.axis_index("c")

    @pl.when(wid == 0)
    def _():
        pltpu.sync_copy(ti_hbm, ti_v)
        pltpu.sync_copy(ci_hbm, ci_v)
        cp_w = pltpu.async_copy(w_hbm.at[ti_v], wrow_v, sem_w)
        cp_c = pltpu.async_copy(c_hbm.at[ci_v], crow_v, sem_c)
        cp_w.wait()
        cp_c.wait()
        acc = jnp.zeros((_LANES,), jnp.float32)
        for j in range(_EMBED // _LANES):
            acc = acc + (wrow_v[0, pl.ds(j * _LANES, _LANES)]
                         * crow_v[0, pl.ds(j * _LANES, _LANES)])
        lanes = lax.iota(jnp.int32, _LANES)
        for sh in (8, 4, 2, 1):
            red_v[...] = acc
            acc = acc + plsc.load_gather(red_v, [lanes ^ sh])
        out_v[...] = 1.0 / (1.0 + jnp.exp(-acc))
        pltpu.sync_copy(out_v, out_hbm)


def kernel(tgt_word, ctx_word, w, c):
    ti = jnp.full((_IPAD,), tgt_word, dtype=jnp.int32)
    ci = jnp.full((_IPAD,), ctx_word, dtype=jnp.int32)
    out = _sc_skipgram(ti, ci, w, c)
    return out[0]


# speedup vs baseline: 1.0822x; 1.0822x over previous
import functools

import jax
import jax.numpy as jnp
from jax import lax
from jax.experimental import pallas as pl
from jax.experimental.pallas import tpu as pltpu
from jax.experimental.pallas import tpu_sc as plsc

_EMBED = 128
_LANES = 16
_IPAD = 8

_mesh = plsc.VectorSubcoreMesh(
    core_axis_name="c", subcore_axis_name="s", num_cores=1, num_subcores=1)


@functools.partial(
    pl.kernel,
    out_type=jax.ShapeDtypeStruct((_LANES,), jnp.float32),
    mesh=_mesh,
    compiler_params=pltpu.CompilerParams(needs_layout_passes=False),
    scratch_types=[
        pltpu.VMEM((_IPAD,), jnp.int32),
        pltpu.VMEM((_IPAD,), jnp.int32),
        pltpu.VMEM((_IPAD, _EMBED), jnp.float32),
        pltpu.VMEM((_IPAD, _EMBED), jnp.float32),
        pltpu.VMEM((_LANES,), jnp.float32),
        pltpu.VMEM((_LANES,), jnp.float32),
        pltpu.SemaphoreType.DMA,
        pltpu.SemaphoreType.DMA,
    ],
)
def _sc_skipgram(ti_hbm, ci_hbm, w_hbm, c_hbm, out_hbm,
                 ti_v, ci_v, wrow_v, crow_v, red_v, out_v, sem_w, sem_c):
    wid = lax.axis_index("s") * 2 + lax.axis_index("c")

    @pl.when(wid == 0)
    def _():
        pltpu.sync_copy(ti_hbm, ti_v)
        pltpu.sync_copy(ci_hbm, ci_v)
        cp_w = pltpu.async_copy(w_hbm.at[ti_v], wrow_v, sem_w)
        cp_c = pltpu.async_copy(c_hbm.at[ci_v], crow_v, sem_c)
        cp_w.wait()
        cp_c.wait()
        acc = jnp.zeros((_LANES,), jnp.float32)
        for j in range(_EMBED // _LANES):
            acc = acc + (wrow_v[0, pl.ds(j * _LANES, _LANES)]
                         * crow_v[0, pl.ds(j * _LANES, _LANES)])
        lanes = lax.iota(jnp.int32, _LANES)
        for sh in (8, 4, 2, 1):
            red_v[...] = acc
            acc = acc + plsc.load_gather(red_v, [lanes ^ sh])
        out_v[...] = 1.0 / (1.0 + jnp.exp(-acc))
        pltpu.sync_copy(out_v, out_hbm)


def kernel(tgt_word, ctx_word, w, c):
    ti = jnp.full((_IPAD,), tgt_word, dtype=jnp.int32)
    ci = jnp.full((_IPAD,), ctx_word, dtype=jnp.int32)
    out = _sc_skipgram(ti, ci, w, c)
    return out[0]


# speedup vs baseline: 1.0988x; 1.0153x over previous
import functools

import jax
import jax.numpy as jnp
from jax import lax
from jax.experimental import pallas as pl
from jax.experimental.pallas import tpu as pltpu
from jax.experimental.pallas import tpu_sc as plsc

_EMBED = 128
_LANES = 16

_mesh = plsc.VectorSubcoreMesh(
    core_axis_name="c", subcore_axis_name="s", num_cores=1, num_subcores=1)


@functools.partial(
    pl.kernel,
    out_type=jax.ShapeDtypeStruct((_LANES,), jnp.float32),
    mesh=_mesh,
    compiler_params=pltpu.CompilerParams(needs_layout_passes=False),
    scratch_types=[
        pltpu.VMEM((1,), jnp.int32),
        pltpu.VMEM((1,), jnp.int32),
        pltpu.VMEM((1, _EMBED), jnp.float32),
        pltpu.VMEM((1, _EMBED), jnp.float32),
        pltpu.VMEM((_LANES,), jnp.float32),
        pltpu.VMEM((_LANES,), jnp.float32),
        pltpu.SemaphoreType.DMA,
        pltpu.SemaphoreType.DMA,
    ],
)
def _sc_skipgram(ti_hbm, ci_hbm, w_hbm, c_hbm, out_hbm,
                 ti_v, ci_v, wrow_v, crow_v, red_v, out_v, sem_w, sem_c):
    cp_ti = pltpu.async_copy(ti_hbm, ti_v, sem_w)
    cp_ci = pltpu.async_copy(ci_hbm, ci_v, sem_c)
    cp_ti.wait()
    cp_ci.wait()
    cp_w = pltpu.async_copy(w_hbm.at[ti_v], wrow_v, sem_w)
    cp_c = pltpu.async_copy(c_hbm.at[ci_v], crow_v, sem_c)
    cp_w.wait()
    cp_c.wait()
    acc = jnp.zeros((_LANES,), jnp.float32)
    for j in range(_EMBED // _LANES):
        acc = acc + (wrow_v[0, pl.ds(j * _LANES, _LANES)]
                     * crow_v[0, pl.ds(j * _LANES, _LANES)])
    lanes = lax.iota(jnp.int32, _LANES)
    for sh in (8, 4, 2, 1):
        red_v[...] = acc
        acc = acc + plsc.load_gather(red_v, [lanes ^ sh])
    out_v[...] = 1.0 / (1.0 + jnp.exp(-acc))
    pltpu.sync_copy(out_v, out_hbm)


def kernel(tgt_word, ctx_word, w, c):
    ti = jnp.reshape(tgt_word.astype(jnp.int32), (1,))
    ci = jnp.reshape(ctx_word.astype(jnp.int32), (1,))
    out = _sc_skipgram(ti, ci, w, c)
    return out[0]


# speedup vs baseline: 1.1258x; 1.0246x over previous
import functools

import jax
import jax.numpy as jnp
from jax import lax
from jax.experimental import pallas as pl
from jax.experimental.pallas import tpu as pltpu
from jax.experimental.pallas import tpu_sc as plsc

_EMBED = 128
_LANES = 16

_mesh = plsc.VectorSubcoreMesh(
    core_axis_name="c", subcore_axis_name="s", num_cores=1, num_subcores=1)


@functools.partial(
    pl.kernel,
    out_type=jax.ShapeDtypeStruct((_LANES,), jnp.float32),
    mesh=_mesh,
    compiler_params=pltpu.CompilerParams(needs_layout_passes=False),
    scratch_types=[
        pltpu.VMEM((1,), jnp.int32),
        pltpu.VMEM((1,), jnp.int32),
        pltpu.VMEM((1, _EMBED), jnp.float32),
        pltpu.VMEM((1, _EMBED), jnp.float32),
        pltpu.VMEM((_LANES,), jnp.float32),
        pltpu.SemaphoreType.DMA,
        pltpu.SemaphoreType.DMA,
    ],
)
def _sc_skipgram(ti_hbm, ci_hbm, w_hbm, c_hbm, out_hbm,
                 ti_v, ci_v, wrow_v, crow_v, out_v, sem_w, sem_c):
    cp_ti = pltpu.async_copy(ti_hbm, ti_v, sem_w)
    cp_ci = pltpu.async_copy(ci_hbm, ci_v, sem_c)
    cp_ti.wait()
    cp_ci.wait()
    cp_w = pltpu.async_copy(w_hbm.at[ti_v], wrow_v, sem_w)
    cp_c = pltpu.async_copy(c_hbm.at[ci_v], crow_v, sem_c)
    cp_w.wait()
    cp_c.wait()
    acc = jnp.zeros((_LANES,), jnp.float32)
    for j in range(_EMBED // _LANES):
        acc = acc + (wrow_v[0, pl.ds(j * _LANES, _LANES)]
                     * crow_v[0, pl.ds(j * _LANES, _LANES)])
    score = jnp.sum(acc)
    sv = jnp.full((_LANES,), score, jnp.float32)
    out_v[...] = 1.0 / (1.0 + jnp.exp(-sv))
    pltpu.sync_copy(out_v, out_hbm)


def kernel(tgt_word, ctx_word, w, c):
    ti = jnp.reshape(tgt_word.astype(jnp.int32), (1,))
    ci = jnp.reshape(ctx_word.astype(jnp.int32), (1,))
    out = _sc_skipgram(ti, ci, w, c)
    return out[0]
